# baseline (speedup 1.0000x reference)
import functools

import jax
import jax.numpy as jnp
from jax.experimental import pallas as pl
from jax.experimental.pallas import tpu as pltpu

MAX_LANES = 14
QUERY_NUM = 80
NUM_SAMPLE_POINTS = 100
ORDER = 3
B = 128
TOPK = 12
NB = 8


def _basis_expanded():
    t = jnp.linspace(0.0, 1.0, NUM_SAMPLE_POINTS)
    k = jnp.arange(ORDER + 1)
    comb = jnp.array([1.0, 3.0, 3.0, 1.0], dtype=jnp.float32)
    basis = comb * (t[:, None] ** k) * ((1.0 - t[:, None]) ** (ORDER - k))
    bE = basis.T[:, None, :, None] * jnp.eye(2, dtype=jnp.float32)[None, :, None, :]
    return bE.reshape(2 * (ORDER + 1), 2 * NUM_SAMPLE_POINTS)


def _tc_body(co_ref, ct_ref, bE_ref, gt_ref, mask_ref):
    bE = bE_ref[...]
    A = jnp.dot(co_ref[...].reshape(NB * MAX_LANES, 8), bE,
                preferred_element_type=jnp.float32)
    T = jnp.dot(ct_ref[...].reshape(NB * QUERY_NUM, 8), bE,
                preferred_element_type=jnp.float32)
    A3 = A.reshape(NB, MAX_LANES, 2 * NUM_SAMPLE_POINTS)
    T3 = T.reshape(NB, QUERY_NUM, 2 * NUM_SAMPLE_POINTS)

    j_idx = jax.lax.broadcasted_iota(jnp.int32, (NB, QUERY_NUM, QUERY_NUM), 1)
    k_idx = jax.lax.broadcasted_iota(jnp.int32, (NB, QUERY_NUM, QUERY_NUM), 2)

    minval = jnp.full((NB, QUERY_NUM), jnp.inf, dtype=jnp.float32)
    minidx = jnp.zeros((NB, QUERY_NUM), dtype=jnp.int32)
    selmask = jnp.zeros((NB, QUERY_NUM), dtype=jnp.float32)

    for i in range(MAX_LANES):
        d = jnp.sum(jnp.abs(A3[:, i, :][:, None, :] - T3), axis=-1)
        dj = d[:, :, None]
        dk = d[:, None, :]
        cmp = (dk < dj) | ((dk == dj) & (k_idx < j_idx))
        rank = jnp.sum(cmp.astype(jnp.int32), axis=-1)
        mask_i = (rank < TOPK).astype(jnp.float32)
        upd = d < minval
        minval = jnp.where(upd, d, minval)
        minidx = jnp.where(upd, i, minidx)
        selmask = jnp.where(upd, mask_i, selmask)

    gt_ref[...] = minidx
    mask_ref[...] = selmask


@jax.jit
def _run_tc(co, ct, bE):
    grid = B // NB
    return pl.pallas_call(
        _tc_body,
        grid=(grid,),
        in_specs=[
            pl.BlockSpec((NB, MAX_LANES, 8), lambda i: (i, 0, 0)),
            pl.BlockSpec((NB, QUERY_NUM, 8), lambda i: (i, 0, 0)),
            pl.BlockSpec((8, 2 * NUM_SAMPLE_POINTS), lambda i: (0, 0)),
        ],
        out_specs=[
            pl.BlockSpec((NB, QUERY_NUM), lambda i: (i, 0)),
            pl.BlockSpec((NB, QUERY_NUM), lambda i: (i, 0)),
        ],
        out_shape=[
            jax.ShapeDtypeStruct((B, QUERY_NUM), jnp.int32),
            jax.ShapeDtypeStruct((B, QUERY_NUM), jnp.float32),
        ],
    )(co, ct, bE)


def kernel(outputs, targets, valid_gt_num):
    Bv = outputs.shape[0]
    co = outputs.reshape(Bv, MAX_LANES, 2 * (ORDER + 1))
    ct = targets.reshape(Bv, QUERY_NUM, 2 * (ORDER + 1))
    bE = _basis_expanded()
    gt_idx, selmask = _run_tc(co, ct, bE)
    re_pred_idx = jnp.broadcast_to(jnp.arange(QUERY_NUM), (Bv, QUERY_NUM))
    return (gt_idx, re_pred_idx, selmask.reshape(-1))


# speedup vs baseline: 33.2733x; 33.2733x over previous
import functools

import jax
import jax.numpy as jnp
from jax import lax
from jax.experimental import pallas as pl
from jax.experimental.pallas import tpu as pltpu
from jax.experimental.pallas import tpu_sc as plsc

MAX_LANES = 14
QUERY_NUM = 80
NUM_SAMPLE_POINTS = 100
ORDER = 3
B = 128
TOPK = 12
NB = 8
BPW = 4


def _basis_expanded():
    t = jnp.linspace(0.0, 1.0, NUM_SAMPLE_POINTS)
    k = jnp.arange(ORDER + 1)
    comb = jnp.array([1.0, 3.0, 3.0, 1.0], dtype=jnp.float32)
    basis = comb * (t[:, None] ** k) * ((1.0 - t[:, None]) ** (ORDER - k))
    bE = basis.T[:, None, :, None] * jnp.eye(2, dtype=jnp.float32)[None, :, None, :]
    return bE.reshape(2 * (ORDER + 1), 2 * NUM_SAMPLE_POINTS)


def _tc_body(co_ref, ct_ref, bE_ref, d_ref):
    bE = bE_ref[...]
    A = jnp.dot(co_ref[...].reshape(NB * MAX_LANES, 8), bE,
                preferred_element_type=jnp.float32)
    T = jnp.dot(ct_ref[...].reshape(NB * QUERY_NUM, 8), bE,
                preferred_element_type=jnp.float32)
    A3 = A.reshape(NB, MAX_LANES, 2 * NUM_SAMPLE_POINTS)
    T3 = T.reshape(NB, QUERY_NUM, 2 * NUM_SAMPLE_POINTS)
    for i in range(MAX_LANES):
        d = jnp.sum(jnp.abs(A3[:, i, :][:, None, :] - T3), axis=-1)
        d_ref[:, i, :] = d


@jax.jit
def _run_tc(co, ct, bE):
    grid = B // NB
    return pl.pallas_call(
        _tc_body,
        grid=(grid,),
        in_specs=[
            pl.BlockSpec((NB, MAX_LANES, 8), lambda i: (i, 0, 0)),
            pl.BlockSpec((NB, QUERY_NUM, 8), lambda i: (i, 0, 0)),
            pl.BlockSpec((8, 2 * NUM_SAMPLE_POINTS), lambda i: (0, 0)),
        ],
        out_specs=pl.BlockSpec((NB, MAX_LANES, QUERY_NUM), lambda i: (i, 0, 0)),
        out_shape=jax.ShapeDtypeStruct((B, MAX_LANES, QUERY_NUM), jnp.float32),
    )(co, ct, bE)


def _vsort(x):
    return plsc.sort_key_val(x, x)[0]


def _sc_body(d_hbm, gt_hbm, mask_hbm, d_v, thr_v, gt_v, mask_v, sem):
    nc = 2
    wid = lax.axis_index("s") * nc + lax.axis_index("c")
    base = wid * BPW
    pltpu.sync_copy(d_hbm.at[pl.ds(base * MAX_LANES * QUERY_NUM,
                                   BPW * MAX_LANES * QUERY_NUM)], d_v)

    for b in range(BPW):
        for i in range(MAX_LANES):
            row = (b * MAX_LANES + i) * QUERY_NUM
            chunks = [_vsort(d_v[pl.ds(row + 16 * j, 16)]) for j in range(5)]
            m = chunks[0]
            for j in range(1, 5):
                m = _vsort(jnp.minimum(m, chunks[j][::-1]))
            thr_v[pl.ds((b * MAX_LANES + i) * 16, 16)] = m
        for j in range(5):
            row0 = b * MAX_LANES * QUERY_NUM + 16 * j
            mv = d_v[pl.ds(row0, 16)]
            mi = jnp.zeros((16,), dtype=jnp.int32)
            for i in range(1, MAX_LANES):
                v = d_v[pl.ds(row0 + i * QUERY_NUM, 16)]
                upd = v < mv
                mv = jnp.where(upd, v, mv)
                mi = jnp.where(upd, i, mi)
            gidx = (b * MAX_LANES + mi) * 16 + 11
            t_sel = plsc.load_gather(thr_v, [gidx])
            sl = pl.ds(b * QUERY_NUM + 16 * j, 16)
            gt_v[sl] = mi
            mask_v[sl] = jnp.where(mv <= t_sel, 1.0, 0.0).astype(jnp.float32)

    pltpu.sync_copy(gt_v, gt_hbm.at[pl.ds(base * QUERY_NUM, BPW * QUERY_NUM)])
    pltpu.sync_copy(mask_v, mask_hbm.at[pl.ds(base * QUERY_NUM, BPW * QUERY_NUM)])


@jax.jit
def _run_sc(d):
    mesh = plsc.VectorSubcoreMesh(core_axis_name="c", subcore_axis_name="s")
    f = pl.kernel(
        _sc_body,
        mesh=mesh,
        compiler_params=pltpu.CompilerParams(needs_layout_passes=False),
        out_type=[
            jax.ShapeDtypeStruct((B * QUERY_NUM,), jnp.int32),
            jax.ShapeDtypeStruct((B * QUERY_NUM,), jnp.float32),
        ],
        scratch_types=[
            pltpu.VMEM((BPW * MAX_LANES * QUERY_NUM,), jnp.float32),
            pltpu.VMEM((BPW * MAX_LANES * 16,), jnp.float32),
            pltpu.VMEM((BPW * QUERY_NUM,), jnp.int32),
            pltpu.VMEM((BPW * QUERY_NUM,), jnp.float32),
            pltpu.SemaphoreType.DMA,
        ],
    )
    return f(d.reshape(-1))


def kernel(outputs, targets, valid_gt_num):
    Bv = outputs.shape[0]
    co = outputs.reshape(Bv, MAX_LANES, 2 * (ORDER + 1))
    ct = targets.reshape(Bv, QUERY_NUM, 2 * (ORDER + 1))
    bE = _basis_expanded()
    d = _run_tc(co, ct, bE)
    gt_idx, selmask = _run_sc(d)
    re_pred_idx = jnp.broadcast_to(jnp.arange(QUERY_NUM), (Bv, QUERY_NUM))
    return (gt_idx.reshape(Bv, QUERY_NUM), re_pred_idx, selmask)


# speedup vs baseline: 34.6794x; 1.0423x over previous
import functools

import jax
import jax.numpy as jnp
import numpy as np
from jax import lax
from jax.experimental import pallas as pl
from jax.experimental.pallas import tpu as pltpu
from jax.experimental.pallas import tpu_sc as plsc

MAX_LANES = 14
QUERY_NUM = 80
NUM_SAMPLE_POINTS = 100
ORDER = 3
B = 128
TOPK = 12
NB = 8
BPW = 4


def _basis_expanded():
    t = jnp.linspace(0.0, 1.0, NUM_SAMPLE_POINTS)
    k = jnp.arange(ORDER + 1)
    comb = jnp.array([1.0, 3.0, 3.0, 1.0], dtype=jnp.float32)
    basis = comb * (t[:, None] ** k) * ((1.0 - t[:, None]) ** (ORDER - k))
    bE = basis.T[:, None, :, None] * jnp.eye(2, dtype=jnp.float32)[None, :, None, :]
    return bE.reshape(2 * (ORDER + 1), 2 * NUM_SAMPLE_POINTS)


_IDX_CONST = np.broadcast_to(np.arange(QUERY_NUM, dtype=np.int32),
                             (B, QUERY_NUM)).copy()


def _tc_body(co_ref, ct_ref, bE_ref, d_ref):
    bE = bE_ref[...]
    A = jnp.dot(co_ref[...].reshape(NB * MAX_LANES, 8), bE,
                preferred_element_type=jnp.float32)
    T = jnp.dot(ct_ref[...].reshape(NB * QUERY_NUM, 8), bE,
                preferred_element_type=jnp.float32)
    A3 = A.reshape(NB, MAX_LANES, 2 * NUM_SAMPLE_POINTS)
    T3 = T.reshape(NB, QUERY_NUM, 2 * NUM_SAMPLE_POINTS)
    for i in range(MAX_LANES):
        d = jnp.sum(jnp.abs(A3[:, i, :][:, None, :] - T3), axis=-1)
        d_ref[:, i, :] = d


@jax.jit
def _run_tc(co, ct, bE):
    grid = B // NB
    return pl.pallas_call(
        _tc_body,
        grid=(grid,),
        in_specs=[
            pl.BlockSpec((NB, MAX_LANES, 8), lambda i: (i, 0, 0)),
            pl.BlockSpec((NB, QUERY_NUM, 8), lambda i: (i, 0, 0)),
            pl.BlockSpec((8, 2 * NUM_SAMPLE_POINTS), lambda i: (0, 0)),
        ],
        out_specs=pl.BlockSpec((NB, MAX_LANES, QUERY_NUM), lambda i: (i, 0, 0)),
        out_shape=jax.ShapeDtypeStruct((B, MAX_LANES, QUERY_NUM), jnp.float32),
    )(co, ct, bE)


def _vsort(x):
    return plsc.sort_key_val(x, x)[0]


def _sc_body(d_hbm, gt_hbm, mask_hbm, d_v, thr_v, gt_v, mask_v, sem):
    nc = 2
    wid = lax.axis_index("s") * nc + lax.axis_index("c")
    base = wid * BPW
    cps = [pltpu.async_copy(
        d_hbm.at[base + b], d_v.at[pl.ds(b * MAX_LANES, MAX_LANES)],
        sem) for b in range(BPW)]
    for c in cps:
        c.wait()

    for b in range(BPW):
        for i in range(MAX_LANES):
            r = b * MAX_LANES + i
            chunks = [_vsort(d_v[r, pl.ds(16 * j, 16)]) for j in range(5)]
            m = chunks[0]
            for j in range(1, 5):
                m = _vsort(jnp.minimum(m, chunks[j][::-1]))
            thr_v[pl.ds(r * 16, 16)] = m
        for j in range(5):
            sl16 = pl.ds(16 * j, 16)
            mv = d_v[b * MAX_LANES, sl16]
            mi = jnp.zeros((16,), dtype=jnp.int32)
            for i in range(1, MAX_LANES):
                v = d_v[b * MAX_LANES + i, sl16]
                upd = v < mv
                mv = jnp.where(upd, v, mv)
                mi = jnp.where(upd, i, mi)
            gidx = (b * MAX_LANES + mi) * 16 + 11
            t_sel = plsc.load_gather(thr_v, [gidx])
            gt_v[b, sl16] = mi
            mask_v[pl.ds(b * QUERY_NUM + 16 * j, 16)] = jnp.where(
                mv <= t_sel, 1.0, 0.0).astype(jnp.float32)

    pltpu.sync_copy(gt_v, gt_hbm.at[pl.ds(base, BPW)])
    pltpu.sync_copy(mask_v, mask_hbm.at[pl.ds(base * QUERY_NUM, BPW * QUERY_NUM)])


@jax.jit
def _run_sc(d):
    mesh = plsc.VectorSubcoreMesh(core_axis_name="c", subcore_axis_name="s")
    f = pl.kernel(
        _sc_body,
        mesh=mesh,
        compiler_params=pltpu.CompilerParams(needs_layout_passes=False),
        out_type=[
            jax.ShapeDtypeStruct((B, QUERY_NUM), jnp.int32),
            jax.ShapeDtypeStruct((B * QUERY_NUM,), jnp.float32),
        ],
        scratch_types=[
            pltpu.VMEM((BPW * MAX_LANES, QUERY_NUM), jnp.float32),
            pltpu.VMEM((BPW * MAX_LANES * 16,), jnp.float32),
            pltpu.VMEM((BPW, QUERY_NUM), jnp.int32),
            pltpu.VMEM((BPW * QUERY_NUM,), jnp.float32),
            pltpu.SemaphoreType.DMA,
        ],
    )
    return f(d)


def kernel(outputs, targets, valid_gt_num):
    Bv = outputs.shape[0]
    co = outputs.reshape(Bv, MAX_LANES, 2 * (ORDER + 1))
    ct = targets.reshape(Bv, QUERY_NUM, 2 * (ORDER + 1))
    bE = _basis_expanded()
    d = _run_tc(co, ct, bE)
    gt_idx, selmask = _run_sc(d)
    return (gt_idx, jnp.asarray(_IDX_CONST), selmask)


# speedup vs baseline: 35.9100x; 1.0355x over previous
import functools

import jax
import jax.numpy as jnp
import numpy as np
from jax import lax
from jax.experimental import pallas as pl
from jax.experimental.pallas import tpu as pltpu
from jax.experimental.pallas import tpu_sc as plsc

MAX_LANES = 14
QUERY_NUM = 80
NUM_SAMPLE_POINTS = 100
ORDER = 3
B = 128
TOPK = 12
NB = 8
BPW = 4


def _basis_expanded():
    t = jnp.linspace(0.0, 1.0, NUM_SAMPLE_POINTS)
    k = jnp.arange(ORDER + 1)
    comb = jnp.array([1.0, 3.0, 3.0, 1.0], dtype=jnp.float32)
    basis = comb * (t[:, None] ** k) * ((1.0 - t[:, None]) ** (ORDER - k))
    bE = basis.T[:, None, :, None] * jnp.eye(2, dtype=jnp.float32)[None, :, None, :]
    return bE.reshape(2 * (ORDER + 1), 2 * NUM_SAMPLE_POINTS)


_IDX_CONST = np.broadcast_to(np.arange(QUERY_NUM, dtype=np.int32),
                             (B, QUERY_NUM)).copy()


def _tc_body(co_ref, ct_ref, bE_ref, d_ref):
    bE = bE_ref[...]
    A = jnp.dot(co_ref[...].reshape(NB * MAX_LANES, 8), bE,
                preferred_element_type=jnp.float32)
    T = jax.lax.dot_general(ct_ref[...], bE, (((0,), (0,)), ((), ())),
                            preferred_element_type=jnp.float32)
    A3 = A.reshape(NB, MAX_LANES, 2 * NUM_SAMPLE_POINTS)
    T3 = T.reshape(NB, QUERY_NUM, 2 * NUM_SAMPLE_POINTS)
    for i in range(MAX_LANES):
        d = jnp.sum(jnp.abs(A3[:, i, :][:, None, :] - T3), axis=-1)
        d_ref[:, i, :] = d


@jax.jit
def _run_tc(co, ct, bE):
    grid = B // NB
    return pl.pallas_call(
        _tc_body,
        grid=(grid,),
        in_specs=[
            pl.BlockSpec((NB, MAX_LANES, 8), lambda i: (i, 0, 0)),
            pl.BlockSpec((8, NB * QUERY_NUM), lambda i: (0, i)),
            pl.BlockSpec((8, 2 * NUM_SAMPLE_POINTS), lambda i: (0, 0)),
        ],
        out_specs=pl.BlockSpec((NB, MAX_LANES, QUERY_NUM), lambda i: (i, 0, 0)),
        out_shape=jax.ShapeDtypeStruct((B, MAX_LANES, QUERY_NUM), jnp.float32),
    )(co, ct, bE)


def _vsort(x):
    return plsc.sort_key_val(x, x)[0]


def _sc_body(d_hbm, gt_hbm, mask_hbm, d_v, thr_v, gt_v, mask_v, sem):
    nc = 2
    wid = lax.axis_index("s") * nc + lax.axis_index("c")
    base = wid * BPW
    cps = [pltpu.async_copy(
        d_hbm.at[base + b], d_v.at[pl.ds(b * MAX_LANES, MAX_LANES)],
        sem) for b in range(BPW)]
    for c in cps:
        c.wait()

    for b in range(BPW):
        for i in range(MAX_LANES):
            r = b * MAX_LANES + i
            chunks = [_vsort(d_v[r, pl.ds(16 * j, 16)]) for j in range(5)]
            m = chunks[0]
            for j in range(1, 5):
                m = _vsort(jnp.minimum(m, chunks[j][::-1]))
            thr_v[pl.ds(r * 16, 16)] = m
        for j in range(5):
            sl16 = pl.ds(16 * j, 16)
            mv = d_v[b * MAX_LANES, sl16]
            mi = jnp.zeros((16,), dtype=jnp.int32)
            for i in range(1, MAX_LANES):
                v = d_v[b * MAX_LANES + i, sl16]
                upd = v < mv
                mv = jnp.where(upd, v, mv)
                mi = jnp.where(upd, i, mi)
            gidx = (b * MAX_LANES + mi) * 16 + 11
            t_sel = plsc.load_gather(thr_v, [gidx])
            gt_v[b, sl16] = mi
            mask_v[pl.ds(b * QUERY_NUM + 16 * j, 16)] = jnp.where(
                mv <= t_sel, 1.0, 0.0).astype(jnp.float32)

    pltpu.sync_copy(gt_v, gt_hbm.at[pl.ds(base, BPW)])
    pltpu.sync_copy(mask_v, mask_hbm.at[pl.ds(base * QUERY_NUM, BPW * QUERY_NUM)])


@jax.jit
def _run_sc(d):
    mesh = plsc.VectorSubcoreMesh(core_axis_name="c", subcore_axis_name="s")
    f = pl.kernel(
        _sc_body,
        mesh=mesh,
        compiler_params=pltpu.CompilerParams(needs_layout_passes=False),
        out_type=[
            jax.ShapeDtypeStruct((B, QUERY_NUM), jnp.int32),
            jax.ShapeDtypeStruct((B * QUERY_NUM,), jnp.float32),
        ],
        scratch_types=[
            pltpu.VMEM((BPW * MAX_LANES, QUERY_NUM), jnp.float32),
            pltpu.VMEM((BPW * MAX_LANES * 16,), jnp.float32),
            pltpu.VMEM((BPW, QUERY_NUM), jnp.int32),
            pltpu.VMEM((BPW * QUERY_NUM,), jnp.float32),
            pltpu.SemaphoreType.DMA,
        ],
    )
    return f(d)


def kernel(outputs, targets, valid_gt_num):
    Bv = outputs.shape[0]
    co = outputs.reshape(Bv, MAX_LANES, 2 * (ORDER + 1))
    ct = targets.reshape(Bv * QUERY_NUM, 2 * (ORDER + 1)).T
    bE = _basis_expanded()
    d = _run_tc(co, ct, bE)
    gt_idx, selmask = _run_sc(d)
    return (gt_idx, jnp.asarray(_IDX_CONST), selmask)


# speedup vs baseline: 37.6351x; 1.0480x over previous
import functools

import jax
import jax.numpy as jnp
import numpy as np
from jax import lax
from jax.experimental import pallas as pl
from jax.experimental.pallas import tpu as pltpu
from jax.experimental.pallas import tpu_sc as plsc

MAX_LANES = 14
QUERY_NUM = 80
NUM_SAMPLE_POINTS = 100
ORDER = 3
B = 128
TOPK = 12
NB = 16
BPW = 4


def _basis_expanded():
    t = jnp.linspace(0.0, 1.0, NUM_SAMPLE_POINTS)
    k = jnp.arange(ORDER + 1)
    comb = jnp.array([1.0, 3.0, 3.0, 1.0], dtype=jnp.float32)
    basis = comb * (t[:, None] ** k) * ((1.0 - t[:, None]) ** (ORDER - k))
    bE = basis.T[:, None, :, None] * jnp.eye(2, dtype=jnp.float32)[None, :, None, :]
    return bE.reshape(2 * (ORDER + 1), 2 * NUM_SAMPLE_POINTS)


_IDX_CONST = np.broadcast_to(np.arange(QUERY_NUM, dtype=np.int32),
                             (B, QUERY_NUM)).copy()


def _tc_body(co_ref, ct_ref, bE_ref, d_ref):
    bE = bE_ref[...]
    A = jnp.dot(co_ref[...].reshape(NB * MAX_LANES, 8), bE,
                preferred_element_type=jnp.float32)
    T = jax.lax.dot_general(ct_ref[...], bE, (((0,), (0,)), ((), ())),
                            preferred_element_type=jnp.float32)
    A3 = A.reshape(NB, MAX_LANES, 2 * NUM_SAMPLE_POINTS)
    T3 = T.reshape(NB, QUERY_NUM, 2 * NUM_SAMPLE_POINTS)
    for i in range(MAX_LANES):
        d = jnp.sum(jnp.abs(A3[:, i, :][:, None, :] - T3), axis=-1)
        d_ref[:, i, :] = d


@jax.jit
def _run_tc(co, ct, bE):
    grid = B // NB
    return pl.pallas_call(
        _tc_body,
        grid=(grid,),
        in_specs=[
            pl.BlockSpec((NB, MAX_LANES, 8), lambda i: (i, 0, 0)),
            pl.BlockSpec((8, NB * QUERY_NUM), lambda i: (0, i)),
            pl.BlockSpec((8, 2 * NUM_SAMPLE_POINTS), lambda i: (0, 0)),
        ],
        out_specs=pl.BlockSpec((NB, MAX_LANES, QUERY_NUM), lambda i: (i, 0, 0)),
        out_shape=jax.ShapeDtypeStruct((B, MAX_LANES, QUERY_NUM), jnp.float32),
    )(co, ct, bE)


def _vsort(x):
    return plsc.sort_key_val(x, x)[0]


def _sc_body(d_hbm, gt_hbm, mask_hbm, d_v, thr_v, gt_v, mask_v, sem):
    nc = 2
    wid = lax.axis_index("s") * nc + lax.axis_index("c")
    base = wid * BPW
    cps = [pltpu.async_copy(
        d_hbm.at[base + b], d_v.at[pl.ds(b * MAX_LANES, MAX_LANES)],
        sem) for b in range(BPW)]
    for c in cps:
        c.wait()

    for b in range(BPW):
        for i in range(MAX_LANES):
            r = b * MAX_LANES + i
            chunks = [_vsort(d_v[r, pl.ds(16 * j, 16)]) for j in range(5)]
            m = chunks[0]
            for j in range(1, 5):
                m = _vsort(jnp.minimum(m, chunks[j][::-1]))
            thr_v[pl.ds(r * 16, 16)] = m
        for j in range(5):
            sl16 = pl.ds(16 * j, 16)
            mv = d_v[b * MAX_LANES, sl16]
            mi = jnp.zeros((16,), dtype=jnp.int32)
            for i in range(1, MAX_LANES):
                v = d_v[b * MAX_LANES + i, sl16]
                upd = v < mv
                mv = jnp.where(upd, v, mv)
                mi = jnp.where(upd, i, mi)
            gidx = (b * MAX_LANES + mi) * 16 + 11
            t_sel = plsc.load_gather(thr_v, [gidx])
            gt_v[b, sl16] = mi
            mask_v[pl.ds(b * QUERY_NUM + 16 * j, 16)] = jnp.where(
                mv <= t_sel, 1.0, 0.0).astype(jnp.float32)

    pltpu.sync_copy(gt_v, gt_hbm.at[pl.ds(base, BPW)])
    pltpu.sync_copy(mask_v, mask_hbm.at[pl.ds(base * QUERY_NUM, BPW * QUERY_NUM)])


@jax.jit
def _run_sc(d):
    mesh = plsc.VectorSubcoreMesh(core_axis_name="c", subcore_axis_name="s")
    f = pl.kernel(
        _sc_body,
        mesh=mesh,
        compiler_params=pltpu.CompilerParams(needs_layout_passes=False),
        out_type=[
            jax.ShapeDtypeStruct((B, QUERY_NUM), jnp.int32),
            jax.ShapeDtypeStruct((B * QUERY_NUM,), jnp.float32),
        ],
        scratch_types=[
            pltpu.VMEM((BPW * MAX_LANES, QUERY_NUM), jnp.float32),
            pltpu.VMEM((BPW * MAX_LANES * 16,), jnp.float32),
            pltpu.VMEM((BPW, QUERY_NUM), jnp.int32),
            pltpu.VMEM((BPW * QUERY_NUM,), jnp.float32),
            pltpu.SemaphoreType.DMA,
        ],
    )
    return f(d)


def kernel(outputs, targets, valid_gt_num):
    Bv = outputs.shape[0]
    co = outputs.reshape(Bv, MAX_LANES, 2 * (ORDER + 1))
    ct = targets.reshape(Bv * QUERY_NUM, 2 * (ORDER + 1)).T
    bE = _basis_expanded()
    d = _run_tc(co, ct, bE)
    gt_idx, selmask = _run_sc(d)
    return (gt_idx, jnp.asarray(_IDX_CONST), selmask)


# speedup vs baseline: 37.9629x; 1.0087x over previous
import functools

import jax
import jax.numpy as jnp
import numpy as np
from jax import lax
from jax.experimental import pallas as pl
from jax.experimental.pallas import tpu as pltpu
from jax.experimental.pallas import tpu_sc as plsc

MAX_LANES = 14
QUERY_NUM = 80
NUM_SAMPLE_POINTS = 100
ORDER = 3
B = 128
TOPK = 12
NB = 32
BPW = 4


def _basis_expanded():
    t = jnp.linspace(0.0, 1.0, NUM_SAMPLE_POINTS)
    k = jnp.arange(ORDER + 1)
    comb = jnp.array([1.0, 3.0, 3.0, 1.0], dtype=jnp.float32)
    basis = comb * (t[:, None] ** k) * ((1.0 - t[:, None]) ** (ORDER - k))
    bE = basis.T[:, None, :, None] * jnp.eye(2, dtype=jnp.float32)[None, :, None, :]
    return bE.reshape(2 * (ORDER + 1), 2 * NUM_SAMPLE_POINTS)


_IDX_CONST = np.broadcast_to(np.arange(QUERY_NUM, dtype=np.int32),
                             (B, QUERY_NUM)).copy()


def _tc_body(co_ref, ct_ref, bE_ref, d_ref):
    bE = bE_ref[...]
    A = jnp.dot(co_ref[...].reshape(NB * MAX_LANES, 8), bE,
                preferred_element_type=jnp.float32)
    T = jax.lax.dot_general(ct_ref[...], bE, (((0,), (0,)), ((), ())),
                            preferred_element_type=jnp.float32)
    A3 = A.reshape(NB, MAX_LANES, 2 * NUM_SAMPLE_POINTS)
    T3 = T.reshape(NB, QUERY_NUM, 2 * NUM_SAMPLE_POINTS)
    for i in range(MAX_LANES):
        d = jnp.sum(jnp.abs(A3[:, i, :][:, None, :] - T3), axis=-1)
        d_ref[:, i, :] = d


@jax.jit
def _run_tc(co, ct, bE):
    grid = B // NB
    return pl.pallas_call(
        _tc_body,
        grid=(grid,),
        in_specs=[
            pl.BlockSpec((NB, MAX_LANES, 8), lambda i: (i, 0, 0)),
            pl.BlockSpec((8, NB * QUERY_NUM), lambda i: (0, i)),
            pl.BlockSpec((8, 2 * NUM_SAMPLE_POINTS), lambda i: (0, 0)),
        ],
        out_specs=pl.BlockSpec((NB, MAX_LANES, QUERY_NUM), lambda i: (i, 0, 0)),
        out_shape=jax.ShapeDtypeStruct((B, MAX_LANES, QUERY_NUM), jnp.float32),
    )(co, ct, bE)


def _vsort(x):
    return plsc.sort_key_val(x, x)[0]


def _sc_body(d_hbm, gt_hbm, mask_hbm, d_v, thr_v, gt_v, mask_v, sem):
    nc = 2
    wid = lax.axis_index("s") * nc + lax.axis_index("c")
    base = wid * BPW
    cps = [pltpu.async_copy(
        d_hbm.at[base + b], d_v.at[pl.ds(b * MAX_LANES, MAX_LANES)],
        sem) for b in range(BPW)]
    for c in cps:
        c.wait()

    for b in range(BPW):
        for i in range(MAX_LANES):
            r = b * MAX_LANES + i
            chunks = [_vsort(d_v[r, pl.ds(16 * j, 16)]) for j in range(5)]
            m = chunks[0]
            for j in range(1, 5):
                m = _vsort(jnp.minimum(m, chunks[j][::-1]))
            thr_v[pl.ds(r * 16, 16)] = m
        for j in range(5):
            sl16 = pl.ds(16 * j, 16)
            mv = d_v[b * MAX_LANES, sl16]
            mi = jnp.zeros((16,), dtype=jnp.int32)
            for i in range(1, MAX_LANES):
                v = d_v[b * MAX_LANES + i, sl16]
                upd = v < mv
                mv = jnp.where(upd, v, mv)
                mi = jnp.where(upd, i, mi)
            gidx = (b * MAX_LANES + mi) * 16 + 11
            t_sel = plsc.load_gather(thr_v, [gidx])
            gt_v[b, sl16] = mi
            mask_v[pl.ds(b * QUERY_NUM + 16 * j, 16)] = jnp.where(
                mv <= t_sel, 1.0, 0.0).astype(jnp.float32)

    pltpu.sync_copy(gt_v, gt_hbm.at[pl.ds(base, BPW)])
    pltpu.sync_copy(mask_v, mask_hbm.at[pl.ds(base * QUERY_NUM, BPW * QUERY_NUM)])


@jax.jit
def _run_sc(d):
    mesh = plsc.VectorSubcoreMesh(core_axis_name="c", subcore_axis_name="s")
    f = pl.kernel(
        _sc_body,
        mesh=mesh,
        compiler_params=pltpu.CompilerParams(needs_layout_passes=False),
        out_type=[
            jax.ShapeDtypeStruct((B, QUERY_NUM), jnp.int32),
            jax.ShapeDtypeStruct((B * QUERY_NUM,), jnp.float32),
        ],
        scratch_types=[
            pltpu.VMEM((BPW * MAX_LANES, QUERY_NUM), jnp.float32),
            pltpu.VMEM((BPW * MAX_LANES * 16,), jnp.float32),
            pltpu.VMEM((BPW, QUERY_NUM), jnp.int32),
            pltpu.VMEM((BPW * QUERY_NUM,), jnp.float32),
            pltpu.SemaphoreType.DMA,
        ],
    )
    return f(d)


def kernel(outputs, targets, valid_gt_num):
    Bv = outputs.shape[0]
    co = outputs.reshape(Bv, MAX_LANES, 2 * (ORDER + 1))
    ct = targets.reshape(Bv * QUERY_NUM, 2 * (ORDER + 1)).T
    bE = _basis_expanded()
    d = _run_tc(co, ct, bE)
    gt_idx, selmask = _run_sc(d)
    return (gt_idx, jnp.asarray(_IDX_CONST), selmask)


# speedup vs baseline: 39.2075x; 1.0328x over previous
import jax
import jax.numpy as jnp
import numpy as np
from jax import lax
from jax.experimental import pallas as pl
from jax.experimental.pallas import tpu as pltpu
from jax.experimental.pallas import tpu_sc as plsc

MAX_LANES = 14
QUERY_NUM = 80
NUM_SAMPLE_POINTS = 100
ORDER = 3
B = 128
TOPK = 12
NB = 64
BPW = 4


def _basis_expanded():
    t = jnp.linspace(0.0, 1.0, NUM_SAMPLE_POINTS)
    k = jnp.arange(ORDER + 1)
    comb = jnp.array([1.0, 3.0, 3.0, 1.0], dtype=jnp.float32)
    basis = comb * (t[:, None] ** k) * ((1.0 - t[:, None]) ** (ORDER - k))
    bE = basis.T[:, None, :, None] * jnp.eye(2, dtype=jnp.float32)[None, :, None, :]
    return bE.reshape(2 * (ORDER + 1), 2 * NUM_SAMPLE_POINTS)


_IDX_CONST = np.broadcast_to(np.arange(QUERY_NUM, dtype=np.int32),
                             (B, QUERY_NUM)).copy()


def _tc_body(co_ref, ct_ref, bE_ref, d_ref):
    bE = bE_ref[...]
    dn = (((0,), (0,)), ((), ()))
    A = jax.lax.dot_general(co_ref[...], bE, dn,
                            preferred_element_type=jnp.float32)
    T = jax.lax.dot_general(ct_ref[...], bE, dn,
                            preferred_element_type=jnp.float32)
    A3 = A.reshape(NB, MAX_LANES, 2 * NUM_SAMPLE_POINTS)
    T3 = T.reshape(NB, QUERY_NUM, 2 * NUM_SAMPLE_POINTS)
    for i in range(MAX_LANES):
        d = jnp.sum(jnp.abs(A3[:, i, :][:, None, :] - T3), axis=-1)
        d_ref[:, i, :] = d


@jax.jit
def _run_tc(co, ct, bE):
    grid = B // NB
    return pl.pallas_call(
        _tc_body,
        grid=(grid,),
        in_specs=[
            pl.BlockSpec((8, NB * MAX_LANES), lambda i: (0, i)),
            pl.BlockSpec((8, NB * QUERY_NUM), lambda i: (0, i)),
            pl.BlockSpec((8, 2 * NUM_SAMPLE_POINTS), lambda i: (0, 0)),
        ],
        out_specs=pl.BlockSpec((NB, MAX_LANES, QUERY_NUM), lambda i: (i, 0, 0)),
        out_shape=jax.ShapeDtypeStruct((B, MAX_LANES, QUERY_NUM), jnp.float32),
    )(co, ct, bE)


def _vsort(x, descending=False):
    return plsc.sort_key_val(x, x, descending=descending)[0]


def _sc_body(d_hbm, gt_hbm, mask_hbm, d_v, thr_v, gt_v, mask_v, sem):
    nc = 2
    wid = lax.axis_index("s") * nc + lax.axis_index("c")
    base = wid * BPW
    cps = [pltpu.async_copy(
        d_hbm.at[base + b], d_v.at[pl.ds(b * MAX_LANES, MAX_LANES)],
        sem) for b in range(BPW)]

    for b in range(BPW):
        cps[b].wait()
        srt = [[_vsort(d_v[b * MAX_LANES + i, pl.ds(16 * j, 16)])
                for j in range(5)] for i in range(MAX_LANES)]
        for i in range(MAX_LANES):
            s = srt[i]
            m01 = _vsort(jnp.minimum(s[0], s[1][::-1]))
            m23 = _vsort(jnp.minimum(s[2], s[3][::-1]))
            m = _vsort(jnp.minimum(m01, m23[::-1]))
            m = _vsort(jnp.minimum(m, s[4][::-1]))
            thr_v[pl.ds((b * MAX_LANES + i) * 16, 16)] = m
        for j in range(5):
            sl16 = pl.ds(16 * j, 16)
            mv = d_v[b * MAX_LANES, sl16]
            mi = jnp.zeros((16,), dtype=jnp.int32)
            for i in range(1, MAX_LANES):
                v = d_v[b * MAX_LANES + i, sl16]
                upd = v < mv
                mv = jnp.where(upd, v, mv)
                mi = jnp.where(upd, i, mi)
            gidx = (b * MAX_LANES + mi) * 16 + (TOPK - 1)
            t_sel = plsc.load_gather(thr_v, [gidx])
            gt_v[b, sl16] = mi
            mask_v[pl.ds(b * QUERY_NUM + 16 * j, 16)] = jnp.where(
                mv <= t_sel, 1.0, 0.0).astype(jnp.float32)

    pltpu.sync_copy(gt_v, gt_hbm.at[pl.ds(base, BPW)])
    pltpu.sync_copy(mask_v, mask_hbm.at[pl.ds(base * QUERY_NUM, BPW * QUERY_NUM)])


@jax.jit
def _run_sc(d):
    mesh = plsc.VectorSubcoreMesh(core_axis_name="c", subcore_axis_name="s")
    f = pl.kernel(
        _sc_body,
        mesh=mesh,
        compiler_params=pltpu.CompilerParams(needs_layout_passes=False),
        out_type=[
            jax.ShapeDtypeStruct((B, QUERY_NUM), jnp.int32),
            jax.ShapeDtypeStruct((B * QUERY_NUM,), jnp.float32),
        ],
        scratch_types=[
            pltpu.VMEM((BPW * MAX_LANES, QUERY_NUM), jnp.float32),
            pltpu.VMEM((BPW * MAX_LANES * 16,), jnp.float32),
            pltpu.VMEM((BPW, QUERY_NUM), jnp.int32),
            pltpu.VMEM((BPW * QUERY_NUM,), jnp.float32),
            pltpu.SemaphoreType.DMA,
        ],
    )
    return f(d)


def kernel(outputs, targets, valid_gt_num):
    Bv = outputs.shape[0]
    co = outputs.reshape(Bv * MAX_LANES, 2 * (ORDER + 1)).T
    ct = targets.reshape(Bv * QUERY_NUM, 2 * (ORDER + 1)).T
    bE = _basis_expanded()
    d = _run_tc(co, ct, bE)
    gt_idx, selmask = _run_sc(d)
    return (gt_idx, jnp.asarray(_IDX_CONST), selmask)
